# baseline (speedup 1.0000x reference)
import functools

import jax
import jax.numpy as jnp
from jax import lax
from jax.experimental import pallas as pl
from jax.experimental.pallas import tpu as pltpu
from jax.experimental.pallas import tpu_sc as plsc

N = 10000
H = 256
B = 128
E = 160000

NC = 2
NS = 16
HALF = N // NC
ACC_ROWS = 5120
ROWS_PER_TILE = ACC_ROWS // NS
LAST_TILE_OUT = HALF - (NS - 1) * ROWS_PER_TILE
EDGES_PER_TILE = E // NS
CHUNK = 48
NBUF = 3
SEG = 2000
NSEG = EDGES_PER_TILE // SEG
MAXC = (SEG + CHUNK - 1) // CHUNK + 1
CBUF = SEG + CHUNK + 16
ZROWS = 8


def _seg_sum_body(x_hbm, src_hbm, dst_hbm, out_hbm,
                  src_all, dst_all, csrc, cdst, gis, sis,
                  rows_bufs, zrows_v, acc, gsems, ssems):
    c = lax.axis_index("c")
    s = lax.axis_index("s")
    base = c * HALF
    dummy = HALF + s

    for r in range(ZROWS):
        for j in range(H // 16):
            zrows_v[r, pl.ds(j * 16, 16)] = jnp.zeros((16,), jnp.float32)
    zcps = []
    for t in range(ROWS_PER_TILE // ZROWS):
        off = pl.multiple_of(s * ROWS_PER_TILE + t * ZROWS, 8)
        zcps.append(pltpu.async_copy(zrows_v, acc.at[pl.ds(off, ZROWS)],
                                     gsems[t % NBUF]))
    for cp in zcps:
        cp.wait()
    plsc.subcore_barrier()

    def segment(q, carry):
        with jax.named_scope("seg_stage"):
            e0 = pl.multiple_of(s * EDGES_PER_TILE + q * SEG, 8)
            pltpu.sync_copy(src_hbm.at[pl.ds(e0, SEG)], src_all)
            pltpu.sync_copy(dst_hbm.at[pl.ds(e0, SEG)], dst_all)

        def compact(g, cnt):
            tr = lax.iota(jnp.int32, 16) + (CBUF - 16)
            off = pl.multiple_of(g * 16, 16)
            dv = dst_all[pl.ds(off, 16)]
            sv = src_all[pl.ds(off, 16)]
            lv = dv - base
            m = (lv >= 0) & (lv < HALF)
            incl = plsc.cumsum(m.astype(jnp.int32))
            pos = jnp.where(m, cnt + incl - 1, tr)
            plsc.store_scatter(csrc, [pos], sv)
            plsc.store_scatter(cdst, [pos], lv)
            return cnt + jnp.max(incl)

        with jax.named_scope("seg_compact"):
            cnt = lax.fori_loop(0, SEG // 16, compact, 0)

        zeros16 = lax.iota(jnp.int32, 16) * 0
        dummy16 = zeros16 + dummy
        for i in range(CHUNK // 16):
            csrc[pl.ds(cnt + i * 16, 16)] = zeros16
            cdst[pl.ds(cnt + i * 16, 16)] = dummy16

        nchunk = (cnt + CHUNK - 1) // CHUNK

        def _drain(b, sem):
            pltpu.make_async_copy(x_hbm.at[pl.ds(0, CHUNK)], rows_bufs[b],
                                  sem).wait()

        with jax.named_scope("seg_pipe"):
            for k in range(MAXC + 1):
                if k < MAXC:
                    b = k % NBUF

                    @pl.when(k < nchunk)
                    def _start_gather(k=k, b=b):
                        if k >= NBUF:
                            _drain(b, ssems[b])
                        for i in range(CHUNK // 16):
                            gis[b][pl.ds(i * 16, 16)] = (
                                csrc[pl.ds(k * CHUNK + i * 16, 16)])
                            sis[b][pl.ds(i * 16, 16)] = (
                                cdst[pl.ds(k * CHUNK + i * 16, 16)])
                        pltpu.async_copy(x_hbm.at[gis[b]], rows_bufs[b],
                                         gsems[b])

                if k >= 1:
                    pb = (k - 1) % NBUF

                    @pl.when(k - 1 < nchunk)
                    def _start_scatter(pb=pb):
                        _drain(pb, gsems[pb])
                        pltpu.async_copy(rows_bufs[pb], acc.at[sis[pb]],
                                         ssems[pb], add=True)

            for b in range(NBUF):
                @pl.when(nchunk > b)
                def _tail(b=b):
                    _drain(b, ssems[b])
        return carry

    lax.fori_loop(0, NSEG, segment, 0)
    plsc.subcore_barrier()

    r0 = pl.multiple_of(s * ROWS_PER_TILE, 8)
    o0 = pl.multiple_of(base + s * ROWS_PER_TILE, 8)

    @pl.when(s < NS - 1)
    def _copy_full():
        pltpu.sync_copy(acc.at[pl.ds(r0, ROWS_PER_TILE)],
                        out_hbm.at[pl.ds(o0, ROWS_PER_TILE)])

    @pl.when(s == NS - 1)
    def _copy_last():
        pltpu.sync_copy(acc.at[pl.ds(r0, LAST_TILE_OUT)],
                        out_hbm.at[pl.ds(o0, LAST_TILE_OUT)])


@functools.cache
def _make_segment_sum():
    return pl.kernel(
        _seg_sum_body,
        out_type=jax.ShapeDtypeStruct((N, H), jnp.float32),
        mesh=plsc.VectorSubcoreMesh(core_axis_name="c", subcore_axis_name="s"),
        compiler_params=pltpu.CompilerParams(use_tc_tiling_on_sc=False,
                                             needs_layout_passes=False),
        scratch_types=[
            pltpu.VMEM((SEG,), jnp.int32),
            pltpu.VMEM((SEG,), jnp.int32),
            pltpu.VMEM((CBUF,), jnp.int32),
            pltpu.VMEM((CBUF,), jnp.int32),
            [pltpu.VMEM((CHUNK,), jnp.int32)] * NBUF,
            [pltpu.VMEM((CHUNK,), jnp.int32)] * NBUF,
            [pltpu.VMEM((CHUNK, H), jnp.float32)] * NBUF,
            pltpu.VMEM((ZROWS, H), jnp.float32),
            pltpu.VMEM_SHARED((ACC_ROWS, H), jnp.float32),
            [pltpu.SemaphoreType.DMA] * NBUF,
            [pltpu.SemaphoreType.DMA] * NBUF,
        ],
    )


def _segment_sum(x, src, dst):
    return _make_segment_sum()(x, src, dst)


ROWS_BLK = 2000


def _mlp_body(eps_ref, x_ref, a_ref, w1_ref, g1_ref, b1_ref, w2_ref,
              d_ref, g2_ref, b2_ref, o_ref):
    e = eps_ref[0]
    t = (1.0 + e) * x_ref[...] + a_ref[...]
    h = jnp.dot(t, w1_ref[...], preferred_element_type=jnp.float32)
    m = jnp.mean(h, axis=1, keepdims=True)
    v = jnp.mean(jnp.square(h - m), axis=1, keepdims=True)
    h = (h - m) * lax.rsqrt(v + 1e-5) * g1_ref[...] + b1_ref[...]
    h = jnp.maximum(h, 0.0)
    u = jnp.dot(h, w2_ref[...], preferred_element_type=jnp.float32)
    u = jnp.maximum(u + d_ref[...], 0.0)
    m2 = jnp.mean(u, axis=1, keepdims=True)
    v2 = jnp.mean(jnp.square(u - m2), axis=1, keepdims=True)
    o_ref[...] = (u - m2) * lax.rsqrt(v2 + 1e-5) * g2_ref[...] + b2_ref[...]


def _mlp_ln(x, agg, eps, W1, ln_g, ln_b, W2, d_row, g2, b2):
    grid = (N // ROWS_BLK,)
    return pl.pallas_call(
        _mlp_body,
        grid=grid,
        in_specs=[
            pl.BlockSpec(memory_space=pltpu.SMEM),
            pl.BlockSpec((ROWS_BLK, H), lambda i: (i, 0)),
            pl.BlockSpec((ROWS_BLK, H), lambda i: (i, 0)),
            pl.BlockSpec((H, B), lambda i: (0, 0)),
            pl.BlockSpec((1, B), lambda i: (0, 0)),
            pl.BlockSpec((1, B), lambda i: (0, 0)),
            pl.BlockSpec((B, H), lambda i: (0, 0)),
            pl.BlockSpec((1, H), lambda i: (0, 0)),
            pl.BlockSpec((1, H), lambda i: (0, 0)),
            pl.BlockSpec((1, H), lambda i: (0, 0)),
        ],
        out_specs=pl.BlockSpec((ROWS_BLK, H), lambda i: (i, 0)),
        out_shape=jax.ShapeDtypeStruct((N, H), jnp.float32),
        compiler_params=pltpu.CompilerParams(
            dimension_semantics=("arbitrary",)),
    )(eps.reshape(1), x, agg, W1, ln_g.reshape(1, B), ln_b.reshape(1, B),
      W2, d_row.reshape(1, H), g2.reshape(1, H), b2.reshape(1, H))


def kernel(x, edge_index, eps_down, Wd1, lnd_g, lnd_b, Wd2, eps_up,
           Wu1, lnu_g, lnu_b, Wu2, ln1_g, ln1_b, ln2_g, ln2_b, dir_emb):
    src = edge_index[0].astype(jnp.int32)
    dst = edge_index[1].astype(jnp.int32)
    agg1 = _segment_sum(x, src, dst)
    x1 = _mlp_ln(x, agg1, eps_down, Wd1, lnd_g, lnd_b, Wd2,
                 dir_emb[0], ln1_g, ln1_b)
    agg2 = _segment_sum(x1, dst, src)
    x2 = _mlp_ln(x1, agg2, eps_up, Wu1, lnu_g, lnu_b, Wu2,
                 dir_emb[1], ln2_g, ln2_b)
    return x2


# speedup vs baseline: 1.1801x; 1.1801x over previous
import functools

import jax
import jax.numpy as jnp
from jax import lax
from jax.experimental import pallas as pl
from jax.experimental.pallas import tpu as pltpu
from jax.experimental.pallas import tpu_sc as plsc

N = 10000
H = 256
B = 128
E = 160000

NC = 2
NS = 16
HALF = N // NC
ACC_ROWS = 5120
ROWS_PER_TILE = ACC_ROWS // NS
LAST_TILE_OUT = HALF - (NS - 1) * ROWS_PER_TILE
EDGES_PER_TILE = E // NS
CHUNK = 16
NBUF = 8
SEG = 2000
NSEG = EDGES_PER_TILE // SEG
MAXC = (SEG + CHUNK - 1) // CHUNK + 1
CBUF = SEG + CHUNK + 16
ZROWS = 8


def _seg_sum_body(x_hbm, src_hbm, dst_hbm, out_hbm,
                  src_all, dst_all, csrc, cdst, gis, sis,
                  rows_bufs, zrows_v, acc, gsems, ssems):
    c = lax.axis_index("c")
    s = lax.axis_index("s")
    base = c * HALF
    dummy = HALF + s

    for r in range(ZROWS):
        for j in range(H // 16):
            zrows_v[r, pl.ds(j * 16, 16)] = jnp.zeros((16,), jnp.float32)
    zcps = []
    for t in range(ROWS_PER_TILE // ZROWS):
        off = pl.multiple_of(s * ROWS_PER_TILE + t * ZROWS, 8)
        zcps.append(pltpu.async_copy(zrows_v, acc.at[pl.ds(off, ZROWS)],
                                     gsems[t % NBUF]))
    for cp in zcps:
        cp.wait()
    plsc.subcore_barrier()

    def segment(q, carry):
        with jax.named_scope("seg_stage"):
            e0 = pl.multiple_of(s * EDGES_PER_TILE + q * SEG, 8)
            pltpu.sync_copy(src_hbm.at[pl.ds(e0, SEG)], src_all)
            pltpu.sync_copy(dst_hbm.at[pl.ds(e0, SEG)], dst_all)

        def compact(g, cnt):
            tr = lax.iota(jnp.int32, 16) + (CBUF - 16)
            off = pl.multiple_of(g * 16, 16)
            dv = dst_all[pl.ds(off, 16)]
            sv = src_all[pl.ds(off, 16)]
            lv = dv - base
            m = (lv >= 0) & (lv < HALF)
            incl = plsc.cumsum(m.astype(jnp.int32))
            pos = jnp.where(m, cnt + incl - 1, tr)
            plsc.store_scatter(csrc, [pos], sv)
            plsc.store_scatter(cdst, [pos], lv)
            return cnt + jnp.max(incl)

        with jax.named_scope("seg_compact"):
            cnt = lax.fori_loop(0, SEG // 16, compact, 0)

        zeros16 = lax.iota(jnp.int32, 16) * 0
        dummy16 = zeros16 + dummy
        for i in range(CHUNK // 16):
            csrc[pl.ds(cnt + i * 16, 16)] = zeros16
            cdst[pl.ds(cnt + i * 16, 16)] = dummy16

        nchunk = (cnt + CHUNK - 1) // CHUNK

        def _drain(b, sem):
            pltpu.make_async_copy(x_hbm.at[pl.ds(0, CHUNK)], rows_bufs[b],
                                  sem).wait()

        with jax.named_scope("seg_pipe"):
            for k in range(MAXC + 1):
                if k < MAXC:
                    b = k % NBUF

                    @pl.when(k < nchunk)
                    def _start_gather(k=k, b=b):
                        if k >= NBUF:
                            _drain(b, ssems[b])
                        for i in range(CHUNK // 16):
                            gis[b][pl.ds(i * 16, 16)] = (
                                csrc[pl.ds(k * CHUNK + i * 16, 16)])
                            sis[b][pl.ds(i * 16, 16)] = (
                                cdst[pl.ds(k * CHUNK + i * 16, 16)])
                        pltpu.async_copy(x_hbm.at[gis[b]], rows_bufs[b],
                                         gsems[b])

                if k >= 1:
                    pb = (k - 1) % NBUF

                    @pl.when(k - 1 < nchunk)
                    def _start_scatter(pb=pb):
                        _drain(pb, gsems[pb])
                        pltpu.async_copy(rows_bufs[pb], acc.at[sis[pb]],
                                         ssems[pb], add=True)

            for b in range(NBUF):
                @pl.when(nchunk > b)
                def _tail(b=b):
                    _drain(b, ssems[b])
        return carry

    lax.fori_loop(0, NSEG, segment, 0)
    plsc.subcore_barrier()

    r0 = pl.multiple_of(s * ROWS_PER_TILE, 8)
    o0 = pl.multiple_of(base + s * ROWS_PER_TILE, 8)

    @pl.when(s < NS - 1)
    def _copy_full():
        pltpu.sync_copy(acc.at[pl.ds(r0, ROWS_PER_TILE)],
                        out_hbm.at[pl.ds(o0, ROWS_PER_TILE)])

    @pl.when(s == NS - 1)
    def _copy_last():
        pltpu.sync_copy(acc.at[pl.ds(r0, LAST_TILE_OUT)],
                        out_hbm.at[pl.ds(o0, LAST_TILE_OUT)])


@functools.cache
def _make_segment_sum():
    return pl.kernel(
        _seg_sum_body,
        out_type=jax.ShapeDtypeStruct((N, H), jnp.float32),
        mesh=plsc.VectorSubcoreMesh(core_axis_name="c", subcore_axis_name="s"),
        compiler_params=pltpu.CompilerParams(use_tc_tiling_on_sc=False,
                                             needs_layout_passes=False),
        scratch_types=[
            pltpu.VMEM((SEG,), jnp.int32),
            pltpu.VMEM((SEG,), jnp.int32),
            pltpu.VMEM((CBUF,), jnp.int32),
            pltpu.VMEM((CBUF,), jnp.int32),
            [pltpu.VMEM((CHUNK,), jnp.int32)] * NBUF,
            [pltpu.VMEM((CHUNK,), jnp.int32)] * NBUF,
            [pltpu.VMEM((CHUNK, H), jnp.float32)] * NBUF,
            pltpu.VMEM((ZROWS, H), jnp.float32),
            pltpu.VMEM_SHARED((ACC_ROWS, H), jnp.float32),
            [pltpu.SemaphoreType.DMA] * NBUF,
            [pltpu.SemaphoreType.DMA] * NBUF,
        ],
    )


def _segment_sum(x, src, dst):
    return _make_segment_sum()(x, src, dst)


ROWS_BLK = 2000


def _mlp_body(eps_ref, x_ref, a_ref, w1_ref, g1_ref, b1_ref, w2_ref,
              d_ref, g2_ref, b2_ref, o_ref):
    e = eps_ref[0]
    t = (1.0 + e) * x_ref[...] + a_ref[...]
    h = jnp.dot(t, w1_ref[...], preferred_element_type=jnp.float32)
    m = jnp.mean(h, axis=1, keepdims=True)
    v = jnp.mean(jnp.square(h - m), axis=1, keepdims=True)
    h = (h - m) * lax.rsqrt(v + 1e-5) * g1_ref[...] + b1_ref[...]
    h = jnp.maximum(h, 0.0)
    u = jnp.dot(h, w2_ref[...], preferred_element_type=jnp.float32)
    u = jnp.maximum(u + d_ref[...], 0.0)
    m2 = jnp.mean(u, axis=1, keepdims=True)
    v2 = jnp.mean(jnp.square(u - m2), axis=1, keepdims=True)
    o_ref[...] = (u - m2) * lax.rsqrt(v2 + 1e-5) * g2_ref[...] + b2_ref[...]


def _mlp_ln(x, agg, eps, W1, ln_g, ln_b, W2, d_row, g2, b2):
    grid = (N // ROWS_BLK,)
    return pl.pallas_call(
        _mlp_body,
        grid=grid,
        in_specs=[
            pl.BlockSpec(memory_space=pltpu.SMEM),
            pl.BlockSpec((ROWS_BLK, H), lambda i: (i, 0)),
            pl.BlockSpec((ROWS_BLK, H), lambda i: (i, 0)),
            pl.BlockSpec((H, B), lambda i: (0, 0)),
            pl.BlockSpec((1, B), lambda i: (0, 0)),
            pl.BlockSpec((1, B), lambda i: (0, 0)),
            pl.BlockSpec((B, H), lambda i: (0, 0)),
            pl.BlockSpec((1, H), lambda i: (0, 0)),
            pl.BlockSpec((1, H), lambda i: (0, 0)),
            pl.BlockSpec((1, H), lambda i: (0, 0)),
        ],
        out_specs=pl.BlockSpec((ROWS_BLK, H), lambda i: (i, 0)),
        out_shape=jax.ShapeDtypeStruct((N, H), jnp.float32),
        compiler_params=pltpu.CompilerParams(
            dimension_semantics=("arbitrary",)),
    )(eps.reshape(1), x, agg, W1, ln_g.reshape(1, B), ln_b.reshape(1, B),
      W2, d_row.reshape(1, H), g2.reshape(1, H), b2.reshape(1, H))


def kernel(x, edge_index, eps_down, Wd1, lnd_g, lnd_b, Wd2, eps_up,
           Wu1, lnu_g, lnu_b, Wu2, ln1_g, ln1_b, ln2_g, ln2_b, dir_emb):
    src = edge_index[0].astype(jnp.int32)
    dst = edge_index[1].astype(jnp.int32)
    agg1 = _segment_sum(x, src, dst)
    x1 = _mlp_ln(x, agg1, eps_down, Wd1, lnd_g, lnd_b, Wd2,
                 dir_emb[0], ln1_g, ln1_b)
    agg2 = _segment_sum(x1, dst, src)
    x2 = _mlp_ln(x1, agg2, eps_up, Wu1, lnu_g, lnu_b, Wu2,
                 dir_emb[1], ln2_g, ln2_b)
    return x2
